# baseline (speedup 1.0000x reference)
import jax
import jax.numpy as jnp
from jax.experimental import pallas as pl


_EPS = 1e-30


def _layer1_body(adj_ref, h_ref, w_ref, out_ref, adjq_ref, colmax_ref):
    i = pl.program_id(0)
    a = adj_ref[...]
    adjq_ref[...] = jnp.round(a * 254.0 - 127.0).astype(jnp.int8)
    t = jnp.dot(a.astype(jnp.bfloat16), h_ref[...],
                preferred_element_type=jnp.float32)
    t = jnp.dot(t, w_ref[...], preferred_element_type=jnp.float32)
    h_out = jnp.maximum(t, 0.0)
    out_ref[...] = h_out.astype(out_ref.dtype)
    m = jnp.broadcast_to(jnp.max(h_out, axis=0, keepdims=True),
                         colmax_ref.shape)

    @pl.when(i == 0)
    def _():
        colmax_ref[...] = m

    @pl.when(i > 0)
    def _():
        colmax_ref[...] = jnp.maximum(colmax_ref[...], m)


def _layer1(adj, h, w, bm):
    m, k = adj.shape
    d = w.shape[1]
    return pl.pallas_call(
        _layer1_body,
        grid=(pl.cdiv(m, bm),),
        in_specs=[
            pl.BlockSpec((bm, k), lambda i: (i, 0)),
            pl.BlockSpec((k, d), lambda i: (0, 0)),
            pl.BlockSpec((d, d), lambda i: (0, 0)),
        ],
        out_specs=[
            pl.BlockSpec((bm, d), lambda i: (i, 0)),
            pl.BlockSpec((bm, k), lambda i: (i, 0)),
            pl.BlockSpec((8, d), lambda i: (0, 0)),
        ],
        out_shape=[
            jax.ShapeDtypeStruct((m, d), jnp.bfloat16),
            jax.ShapeDtypeStruct((m, k), jnp.int8),
            jax.ShapeDtypeStruct((8, d), jnp.float32),
        ],
    )(adj, h, w)


def _quant_body(h_ref, colmax_ref, p_ref, r_ref, cs_ref):
    i = pl.program_id(0)
    s = jnp.maximum(colmax_ref[0:1, :], _EPS) * (1.0 / 127.0)
    inv = 1.0 / s
    hf = h_ref[...].astype(jnp.float32)
    p = jnp.round(hf * inv)
    r = jnp.round((hf - p * s) * (254.0 * inv))
    p_ref[...] = p.astype(jnp.int8)
    r_ref[...] = r.astype(jnp.int8)
    csp = jnp.sum(p, axis=0, keepdims=True)
    csr = jnp.sum(r, axis=0, keepdims=True)
    row = jnp.concatenate(
        [csp, csr, jnp.zeros((6, csp.shape[1]), jnp.float32)], axis=0)

    @pl.when(i == 0)
    def _():
        cs_ref[...] = row

    @pl.when(i > 0)
    def _():
        cs_ref[...] = cs_ref[...] + row


def _quantize(h, colmax, bm):
    n, d = h.shape
    return pl.pallas_call(
        _quant_body,
        grid=(pl.cdiv(n, bm),),
        in_specs=[
            pl.BlockSpec((bm, d), lambda i: (i, 0)),
            pl.BlockSpec((8, d), lambda i: (0, 0)),
        ],
        out_specs=[
            pl.BlockSpec((bm, d), lambda i: (i, 0)),
            pl.BlockSpec((bm, d), lambda i: (i, 0)),
            pl.BlockSpec((8, d), lambda i: (0, 0)),
        ],
        out_shape=[
            jax.ShapeDtypeStruct((n, d), jnp.int8),
            jax.ShapeDtypeStruct((n, d), jnp.int8),
            jax.ShapeDtypeStruct((8, d), jnp.float32),
        ],
    )(h, colmax)


def _qlayer_mid_body(adjq_ref, p_ref, r_ref, colmax_ref, cs_ref, w_ref,
                     out_ref, colmax_out_ref):
    i = pl.program_id(0)
    h_out = _qlayer_compute(adjq_ref, p_ref, r_ref, colmax_ref, cs_ref, w_ref)
    out_ref[...] = h_out.astype(out_ref.dtype)
    m = jnp.broadcast_to(jnp.max(h_out, axis=0, keepdims=True),
                         colmax_out_ref.shape)

    @pl.when(i == 0)
    def _():
        colmax_out_ref[...] = m

    @pl.when(i > 0)
    def _():
        colmax_out_ref[...] = jnp.maximum(colmax_out_ref[...], m)


def _qlayer_last_body(adjq_ref, p_ref, r_ref, colmax_ref, cs_ref, w_ref,
                      out_ref):
    h_out = _qlayer_compute(adjq_ref, p_ref, r_ref, colmax_ref, cs_ref, w_ref)
    out_ref[...] = h_out.astype(out_ref.dtype)


def _qlayer_compute(adjq_ref, p_ref, r_ref, colmax_ref, cs_ref, w_ref):
    q = adjq_ref[...]
    t1 = jnp.dot(q, p_ref[...], preferred_element_type=jnp.int32)
    t2 = jnp.dot(q, r_ref[...], preferred_element_type=jnp.int32)
    s = jnp.maximum(colmax_ref[0:1, :], _EPS) * (1.0 / 127.0)
    corr = 127.0 * (cs_ref[0:1, :] + cs_ref[1:2, :] * (1.0 / 254.0))
    t = (t1.astype(jnp.float32) + t2.astype(jnp.float32) * (1.0 / 254.0)
         + corr) * (s * (1.0 / 254.0))
    t = jnp.dot(t, w_ref[...], preferred_element_type=jnp.float32)
    return jnp.maximum(t, 0.0)


def _qlayer(adjq, p, r, colmax, cs, w, out_dtype, bm, last):
    m, k = adjq.shape
    d = w.shape[1]
    in_specs = [
        pl.BlockSpec((bm, k), lambda i: (i, 0)),
        pl.BlockSpec((k, d), lambda i: (0, 0)),
        pl.BlockSpec((k, d), lambda i: (0, 0)),
        pl.BlockSpec((8, d), lambda i: (0, 0)),
        pl.BlockSpec((8, d), lambda i: (0, 0)),
        pl.BlockSpec((d, d), lambda i: (0, 0)),
    ]
    if last:
        return pl.pallas_call(
            _qlayer_last_body,
            grid=(pl.cdiv(m, bm),),
            in_specs=in_specs,
            out_specs=pl.BlockSpec((bm, d), lambda i: (i, 0)),
            out_shape=jax.ShapeDtypeStruct((m, d), out_dtype),
        )(adjq, p, r, colmax, cs, w)
    return pl.pallas_call(
        _qlayer_mid_body,
        grid=(pl.cdiv(m, bm),),
        in_specs=in_specs,
        out_specs=[
            pl.BlockSpec((bm, d), lambda i: (i, 0)),
            pl.BlockSpec((8, d), lambda i: (0, 0)),
        ],
        out_shape=[
            jax.ShapeDtypeStruct((m, d), out_dtype),
            jax.ShapeDtypeStruct((8, d), jnp.float32),
        ],
    )(adjq, p, r, colmax, cs, w)


def kernel(x, adj, W0, W1, W2):
    h1, adjq, cmax1 = _layer1(adj, x.astype(jnp.bfloat16), W0, 400)
    p1, r1, cs1 = _quantize(h1, cmax1, 1000)
    h2, cmax2 = _qlayer(adjq, p1, r1, cmax1, cs1, W1, jnp.bfloat16, 1000,
                        last=False)
    p2, r2, cs2 = _quantize(h2, cmax2, 1000)
    return _qlayer(adjq, p2, r2, cmax2, cs2, W2, jnp.float32, 1000, last=True)


# speedup vs baseline: 1.2320x; 1.2320x over previous
import jax
import jax.numpy as jnp
from jax.experimental import pallas as pl


def _layer1_body(adj_ref, h_ref, w_ref, out_ref, adj16_ref):
    a16 = adj_ref[...].astype(jnp.bfloat16)
    adj16_ref[...] = a16
    t = jnp.dot(a16, h_ref[...], preferred_element_type=jnp.float32)
    t = jnp.dot(t, w_ref[...], preferred_element_type=jnp.float32)
    out_ref[...] = jnp.maximum(t, 0.0).astype(out_ref.dtype)


def _layer1(adj, h, w, bm):
    m, k = adj.shape
    d = w.shape[1]
    return pl.pallas_call(
        _layer1_body,
        grid=(pl.cdiv(m, bm),),
        in_specs=[
            pl.BlockSpec((bm, k), lambda i: (i, 0)),
            pl.BlockSpec((k, d), lambda i: (0, 0)),
            pl.BlockSpec((d, d), lambda i: (0, 0)),
        ],
        out_specs=[
            pl.BlockSpec((bm, d), lambda i: (i, 0)),
            pl.BlockSpec((bm, k), lambda i: (i, 0)),
        ],
        out_shape=[
            jax.ShapeDtypeStruct((m, d), jnp.bfloat16),
            jax.ShapeDtypeStruct((m, k), jnp.bfloat16),
        ],
    )(adj, h, w)


def _layer_body(adj_ref, h_ref, w_ref, out_ref):
    t = jnp.dot(adj_ref[...], h_ref[...], preferred_element_type=jnp.float32)
    t = jnp.dot(t, w_ref[...], preferred_element_type=jnp.float32)
    out_ref[...] = jnp.maximum(t, 0.0).astype(out_ref.dtype)


def _layer(adj, h, w, out_dtype, bm):
    m, k = adj.shape
    d = w.shape[1]
    return pl.pallas_call(
        _layer_body,
        grid=(pl.cdiv(m, bm),),
        in_specs=[
            pl.BlockSpec((bm, k), lambda i: (i, 0)),
            pl.BlockSpec((k, d), lambda i: (0, 0)),
            pl.BlockSpec((d, d), lambda i: (0, 0)),
        ],
        out_specs=pl.BlockSpec((bm, d), lambda i: (i, 0)),
        out_shape=jax.ShapeDtypeStruct((m, d), out_dtype),
    )(adj, h, w)


def kernel(x, adj, W0, W1, W2):
    h, adj16 = _layer1(adj, x.astype(jnp.bfloat16), W0, 400)
    h = _layer(adj16, h, W1, jnp.bfloat16, 1000)
    return _layer(adj16, h, W2, jnp.float32, 1000)


# speedup vs baseline: 1.2336x; 1.0013x over previous
import jax
import jax.numpy as jnp
from jax.experimental import pallas as pl


def _layer1_body(adj_ref, x_ref, w0_ref, w1_ref, out_ref, adj16_ref):
    a16 = adj_ref[...].astype(jnp.bfloat16)
    adj16_ref[...] = a16
    t = jnp.dot(a16, x_ref[...], preferred_element_type=jnp.float32)
    h = jnp.maximum(
        jnp.dot(t, w0_ref[...], preferred_element_type=jnp.float32), 0.0)
    out_ref[...] = jnp.dot(h, w1_ref[...],
                           preferred_element_type=jnp.float32
                           ).astype(out_ref.dtype)


def _layer1(adj, x, w0, w1, bm):
    m, k = adj.shape
    d = x.shape[1]
    return pl.pallas_call(
        _layer1_body,
        grid=(pl.cdiv(m, bm),),
        in_specs=[
            pl.BlockSpec((bm, k), lambda i: (i, 0)),
            pl.BlockSpec((k, d), lambda i: (0, 0)),
            pl.BlockSpec((d, d), lambda i: (0, 0)),
            pl.BlockSpec((d, d), lambda i: (0, 0)),
        ],
        out_specs=[
            pl.BlockSpec((bm, d), lambda i: (i, 0)),
            pl.BlockSpec((bm, k), lambda i: (i, 0)),
        ],
        out_shape=[
            jax.ShapeDtypeStruct((m, d), jnp.bfloat16),
            jax.ShapeDtypeStruct((m, k), jnp.bfloat16),
        ],
    )(adj, x, w0, w1)


def _layer2_body(adj_ref, s_ref, w2_ref, out_ref):
    t = jnp.dot(adj_ref[...], s_ref[...], preferred_element_type=jnp.float32)
    h = jnp.maximum(t, 0.0)
    out_ref[...] = jnp.dot(h, w2_ref[...],
                           preferred_element_type=jnp.float32
                           ).astype(out_ref.dtype)


def _layer2(adj, s, w2, bm):
    m, k = adj.shape
    d = s.shape[1]
    return pl.pallas_call(
        _layer2_body,
        grid=(pl.cdiv(m, bm),),
        in_specs=[
            pl.BlockSpec((bm, k), lambda i: (i, 0)),
            pl.BlockSpec((k, d), lambda i: (0, 0)),
            pl.BlockSpec((d, d), lambda i: (0, 0)),
        ],
        out_specs=pl.BlockSpec((bm, d), lambda i: (i, 0)),
        out_shape=jax.ShapeDtypeStruct((m, d), jnp.bfloat16),
    )(adj, s, w2)


def _layer3_body(adj_ref, s_ref, out_ref):
    t = jnp.dot(adj_ref[...], s_ref[...], preferred_element_type=jnp.float32)
    out_ref[...] = jnp.maximum(t, 0.0)


def _layer3(adj, s, bm):
    m, k = adj.shape
    d = s.shape[1]
    return pl.pallas_call(
        _layer3_body,
        grid=(pl.cdiv(m, bm),),
        in_specs=[
            pl.BlockSpec((bm, k), lambda i: (i, 0)),
            pl.BlockSpec((k, d), lambda i: (0, 0)),
        ],
        out_specs=pl.BlockSpec((bm, d), lambda i: (i, 0)),
        out_shape=jax.ShapeDtypeStruct((m, d), jnp.float32),
    )(adj, s)


def kernel(x, adj, W0, W1, W2):
    s1, adj16 = _layer1(adj, x.astype(jnp.bfloat16), W0, W1, 400)
    s2 = _layer2(adj16, s1, W2, 1000)
    return _layer3(adj16, s2, 1000)
